# Initial kernel scaffold; baseline (speedup 1.0000x reference)
import jax
import jax.numpy as jnp
from jax.experimental import pallas as pl


def kernel(center_coords, group_features, W1, b1, W2, b2):
    raise NotImplementedError("write your pallas kernel here")


# speedup vs baseline: 580.1656x; 580.1656x over previous
import jax
import jax.numpy as jnp
from jax.experimental import pallas as pl
from jax.experimental.pallas import tpu as pltpu

B, G, C, H = 16, 256, 128, 256
TAU, SINKHORN_ITERS = 0.1, 10


def _lse_last(x):
    m = jnp.max(x, axis=-1, keepdims=True)
    return m + jnp.log(jnp.sum(jnp.exp(x - m), axis=-1, keepdims=True))


def _lse_sub(x):
    m = jnp.max(x, axis=-2, keepdims=True)
    return m + jnp.log(jnp.sum(jnp.exp(x - m), axis=-2, keepdims=True))


def _tc_body(cc_ref, gf_ref, w1_ref, b1_ref, w2_ref, b2_ref,
             rc_ref, rf_ref, perm_ref):
    gf = gf_ref[0]
    cc = cc_ref[0]
    h = jnp.maximum(
        jnp.dot(gf, w1_ref[...], preferred_element_type=jnp.float32)
        + b1_ref[...], 0.0)
    scores = jnp.dot(h, w2_ref[...], preferred_element_type=jnp.float32) \
        + b2_ref[...]

    la = scores / TAU

    def sk(_, la):
        la = la - _lse_last(la)
        la = la - _lse_sub(la)
        return la

    la = jax.lax.fori_loop(0, SINKHORN_ITERS, sk, la)
    P = jnp.exp(la)

    col_ids = jax.lax.broadcasted_iota(jnp.int32, (G, G), 1)
    row_ids = jax.lax.broadcasted_iota(jnp.int32, (G, G), 0)

    def cond(state):
        n, rounds, M, rowfree, colfree = state
        return (n < G) & (rounds < G + 2)

    def body(state):
        n, rounds, M, rowfree, colfree = state
        free = (rowfree > 0.5) & (colfree > 0.5)
        A = jnp.where(free, P, -1.0)
        rmax = jnp.max(A, axis=1, keepdims=True)
        rarg = jnp.min(jnp.where(A == rmax, col_ids, G), axis=1, keepdims=True)
        cmax = jnp.max(A, axis=0, keepdims=True)
        carg = jnp.min(jnp.where(A == cmax, row_ids, G), axis=0, keepdims=True)
        matched = (col_ids == rarg) & (row_ids == carg) & (A > -0.5)
        mf = matched.astype(jnp.float32)
        M = M + mf
        nr = jnp.sum(mf)
        rowfree = rowfree - jnp.sum(mf, axis=1, keepdims=True)
        colfree = colfree - jnp.sum(mf, axis=0, keepdims=True)
        return (n + nr.astype(jnp.int32), rounds + 1, M, rowfree, colfree)

    init = (jnp.int32(0), jnp.int32(0),
            jnp.zeros((G, G), jnp.float32),
            jnp.ones((G, 1), jnp.float32),
            jnp.ones((1, G), jnp.float32))
    _, _, M, _, _ = jax.lax.while_loop(cond, body, init)

    contract = (((0,), (0,)), ((), ()))
    rf_ref[0] = jax.lax.dot_general(M, gf, contract,
                                    preferred_element_type=jnp.float32)
    rc_ref[0] = jax.lax.dot_general(M, cc, contract,
                                    preferred_element_type=jnp.float32)
    cvals = jax.lax.broadcasted_iota(jnp.int32, (G, 1), 0).astype(jnp.float32)
    perm_ref[0] = jnp.dot(M, cvals,
                          preferred_element_type=jnp.float32).astype(jnp.int32)


def kernel(center_coords, group_features, W1, b1, W2, b2):
    b1r = b1.reshape(1, H)
    b2r = b2.reshape(1, G)
    rc, rf, perm = pl.pallas_call(
        _tc_body,
        grid=(B,),
        in_specs=[
            pl.BlockSpec((1, G, 3), lambda b: (b, 0, 0)),
            pl.BlockSpec((1, G, C), lambda b: (b, 0, 0)),
            pl.BlockSpec((C, H), lambda b: (0, 0)),
            pl.BlockSpec((1, H), lambda b: (0, 0)),
            pl.BlockSpec((H, G), lambda b: (0, 0)),
            pl.BlockSpec((1, G), lambda b: (0, 0)),
        ],
        out_specs=[
            pl.BlockSpec((1, G, 3), lambda b: (b, 0, 0)),
            pl.BlockSpec((1, G, C), lambda b: (b, 0, 0)),
            pl.BlockSpec((1, G, 1), lambda b: (b, 0, 0)),
        ],
        out_shape=[
            jax.ShapeDtypeStruct((B, G, 3), jnp.float32),
            jax.ShapeDtypeStruct((B, G, C), jnp.float32),
            jax.ShapeDtypeStruct((B, G, 1), jnp.int32),
        ],
    )(center_coords, group_features, W1, b1r, W2, b2r)
    return (rc, rf, perm.reshape(B, G))
